# Initial kernel scaffold; baseline (speedup 1.0000x reference)
import jax
import jax.numpy as jnp
from jax.experimental import pallas as pl


def kernel(w_all, b_all, x):
    raise NotImplementedError("write your pallas kernel here")


# speedup vs baseline: 1.5167x; 1.5167x over previous
import functools

import jax
import jax.numpy as jnp
from jax.experimental import pallas as pl
from jax.experimental.pallas import tpu as pltpu

_FACTOR = 2
_SD = 8


def _step_kernel(x_ref, w_ref, b_ref, out_ref, *, W, T, L, F2, SD, FILT):
    HW = x_ref.shape[-1]
    C3 = 3 * F2
    n_real = 9 + C3 + SD

    p = jax.lax.broadcasted_iota(jnp.int32, (1, HW), 1)
    col = p % W
    row_top = p >= W
    row_bot = p < HW - W
    col_l = col >= 1
    col_r = col <= W - 2
    zrows = (jnp.zeros((FILT - n_real, HW), jnp.float32)
             if FILT > n_real else None)

    def conv(a_f32, layer, relu):
        am = jnp.where(row_top, pltpu.roll(a_f32, W, axis=1), 0.0)
        ap = jnp.where(row_bot, pltpu.roll(a_f32, HW - W, axis=1), 0.0)
        xrows = jnp.concatenate(
            [am.astype(jnp.bfloat16),
             a_f32.astype(jnp.bfloat16),
             ap.astype(jnp.bfloat16)], axis=0)
        z = jnp.dot(w_ref[layer], xrows,
                    preferred_element_type=jnp.float32)
        zl = pltpu.roll(z[:FILT], 1, axis=1)
        zr = pltpu.roll(z[2 * FILT:], HW - 1, axis=1)
        y = (z[FILT:2 * FILT] + b_ref[layer][:, :1]
             + jnp.where(col_l, zl, 0.0)
             + jnp.where(col_r, zr, 0.0))
        return jnp.maximum(y, 0.0) if relu else y

    def step(t, carry):
        fb, st = carry
        tp = jnp.maximum(t - 1, 0)
        tn = jnp.minimum(t + 1, T - 1)
        f_cur = x_ref[t]

        pieces = [x_ref[tp], f_cur, x_ref[tn], fb, st]
        if zrows is not None:
            pieces.append(zrows)
        a = jnp.concatenate(pieces, axis=0)

        for l in range(L - 1):
            a = conv(a, l, relu=True)
        y = conv(a, L - 1, relu=False)

        rgb = y[:C3] + jnp.concatenate([f_cur] * F2, axis=0)
        st_new = jnp.maximum(y[C3:C3 + SD], 0.0)
        out_ref[t] = rgb
        return rgb, st_new

    fb0 = jnp.zeros((C3, HW), jnp.float32)
    st0 = jnp.zeros((SD, HW), jnp.float32)
    jax.lax.fori_loop(0, T, step, (fb0, st0))


@jax.jit
def _forward(w_all, b_all, x):
    B, T, C, H, W = x.shape
    f = _FACTOR
    F2 = f * f
    SD = _SD
    L, _, FILT, _ = w_all.shape
    HW = H * W

    x_r = x.reshape(B, T, C, HW)
    w_r = w_all.reshape(L, 3, 3, FILT, FILT)
    w_stack = jnp.transpose(w_r, (0, 2, 3, 1, 4)).reshape(L, 3 * FILT, 3 * FILT)
    b_rep = jnp.broadcast_to(b_all[:, :, None], (L, FILT, 128)).astype(jnp.float32)

    kernel_fn = functools.partial(_step_kernel, W=W, T=T, L=L, F2=F2, SD=SD,
                                  FILT=FILT)

    out_flat = pl.pallas_call(
        kernel_fn,
        out_shape=jax.ShapeDtypeStruct((B, T, 3 * F2, HW), jnp.float32),
        grid=(B,),
        in_specs=[
            pl.BlockSpec((None, T, C, HW), lambda b: (b, 0, 0, 0)),
            pl.BlockSpec((L, 3 * FILT, 3 * FILT), lambda b: (0, 0, 0)),
            pl.BlockSpec((L, FILT, 128), lambda b: (0, 0, 0)),
        ],
        out_specs=pl.BlockSpec((None, T, 3 * F2, HW), lambda b: (b, 0, 0, 0)),
        compiler_params=pltpu.CompilerParams(
            dimension_semantics=("parallel",)),
    )(x_r, w_stack, b_rep)

    y = out_flat.reshape(B, T, f, f, C, H, W)
    y = jnp.transpose(y, (0, 1, 4, 5, 2, 6, 3))
    return y.reshape(B, T, C, f * H, f * W)


def kernel(w_all, b_all, x):
    return _forward(w_all, b_all, x)
